# Initial kernel scaffold; baseline (speedup 1.0000x reference)
import jax
import jax.numpy as jnp
from jax.experimental import pallas as pl


def kernel(node_embeddings, proj1_weights, proj1_bias, ln1_weight, ln1_bias, proj2_weights, proj2_bias, ln2_weight, ln2_bias, edge_index, batch):
    raise NotImplementedError("write your pallas kernel here")


# speedup vs baseline: 8.7101x; 8.7101x over previous
import jax
import jax.numpy as jnp
from jax import lax
from jax.experimental import pallas as pl
from jax.experimental.pallas import tpu as pltpu

_NUM_HEADS = 8
_D = 128
_ROWS = 88
_CHUNK = 8
_NCHUNK = _ROWS // _CHUNK
_WCHUNKS = 4
_B = 2048
_N = 524288
_G = 64
_NBLK = _B // _G
_H1 = 256
_H2 = 128
_EPS = 1e-5


def _tc_body(offs_ref, emb_hbm, w1_ref, b1_ref, w2_ref, b2_ref,
             ln1w_ref, ln1b_ref, ln2w_ref, ln2b_ref,
             out_ref, ebuf, sem):
    blk = pl.program_id(0)

    for g in range(_G):
        off = offs_ref[blk * _G + g]
        pltpu.make_async_copy(
            emb_hbm.at[pl.ds(off, _ROWS)],
            ebuf.at[pl.ds(g * _ROWS, _ROWS)],
            sem,
        ).start()
    pltpu.make_async_copy(
        emb_hbm.at[pl.ds(0, _G * _ROWS)], ebuf, sem
    ).wait()

    e = ebuf[...]
    c = e.reshape(_G * _NCHUNK, _CHUNK, _D)
    csum = jnp.sum(c, axis=1).reshape(_G, _NCHUNK, _D)
    cmax = jnp.max(c, axis=1).reshape(_G, _NCHUNK, _D)

    for h in range(_NUM_HEADS):
        s = jnp.sum(csum[:, h:h + _WCHUNKS, :], axis=1)
        m = jnp.max(cmax[:, h:h + _WCHUNKS, :], axis=1)
        x = jnp.concatenate([s * (1.0 / 32.0), m, s], axis=-1)

        y = jnp.dot(x, w1_ref[h], preferred_element_type=jnp.float32)
        y = y + b1_ref[h:h + 1, :]
        mu = jnp.mean(y, axis=-1, keepdims=True)
        var = jnp.mean(jnp.square(y - mu), axis=-1, keepdims=True)
        y = (y - mu) * lax.rsqrt(var + _EPS) * ln1w_ref[...] + ln1b_ref[...]
        y = jnp.maximum(y, 0.0)

        z = jnp.dot(y, w2_ref[h], preferred_element_type=jnp.float32)
        z = z + b2_ref[h:h + 1, :]
        mu2 = jnp.mean(z, axis=-1, keepdims=True)
        var2 = jnp.mean(jnp.square(z - mu2), axis=-1, keepdims=True)
        z = (z - mu2) * lax.rsqrt(var2 + _EPS) * ln2w_ref[...] + ln2b_ref[...]
        out_ref[:, h, :] = jnp.maximum(z, 0.0)


def _run_tc(offsets, node_embeddings, proj1_weights, proj1_bias,
            proj2_weights, proj2_bias, ln1w, ln1b, ln2w, ln2b):
    full = lambda shape: pl.BlockSpec(shape, lambda i: (0,) * len(shape))
    return pl.pallas_call(
        _tc_body,
        grid=(_NBLK,),
        in_specs=[
            pl.BlockSpec(memory_space=pltpu.SMEM),
            pl.BlockSpec(memory_space=pltpu.HBM),
            full((_NUM_HEADS, 3 * _D, _H1)),
            full((_NUM_HEADS, _H1)),
            full((_NUM_HEADS, _H1, _H2)),
            full((_NUM_HEADS, _H2)),
            full((1, _H1)),
            full((1, _H1)),
            full((1, _H2)),
            full((1, _H2)),
        ],
        out_specs=pl.BlockSpec((_G, _NUM_HEADS, _H2), lambda i: (i, 0, 0)),
        out_shape=jax.ShapeDtypeStruct((_B, _NUM_HEADS, _H2), jnp.float32),
        scratch_shapes=[
            pltpu.VMEM((_G * _ROWS, _D), jnp.float32),
            pltpu.SemaphoreType.DMA,
        ],
        compiler_params=pltpu.CompilerParams(
            dimension_semantics=("arbitrary",),
        ),
    )(offsets, node_embeddings, proj1_weights, proj1_bias,
      proj2_weights, proj2_bias, ln1w, ln1b, ln2w, ln2b)


def kernel(node_embeddings, proj1_weights, proj1_bias, ln1_weight, ln1_bias,
           proj2_weights, proj2_bias, ln2_weight, ln2_bias, edge_index, batch):
    del edge_index
    graphs = jnp.arange(_B, dtype=batch.dtype)
    offsets = jnp.searchsorted(batch, graphs, side="left").astype(jnp.int32)
    offsets = jnp.minimum(offsets, _N - _ROWS)
    return _run_tc(
        offsets, node_embeddings, proj1_weights, proj1_bias,
        proj2_weights, proj2_bias,
        ln1_weight.reshape(1, _H1), ln1_bias.reshape(1, _H1),
        ln2_weight.reshape(1, _H2), ln2_bias.reshape(1, _H2),
    )


# speedup vs baseline: 15.9901x; 1.8358x over previous
import functools

import jax
import jax.numpy as jnp
from jax import lax
from jax.experimental import pallas as pl
from jax.experimental.pallas import tpu as pltpu
from jax.experimental.pallas import tpu_sc as plsc

_NUM_HEADS = 8
_D = 128
_ROWS = 88
_CHUNK = 8
_NCHUNK = _ROWS // _CHUNK
_WCHUNKS = 4
_B = 2048
_N = 524288
_G = 64
_NBLK = _B // _G
_H1 = 256
_H2 = 128
_EPS = 1e-5


_SC_NC = 2
_SC_NS = 16
_SC_NW = _SC_NC * _SC_NS
_GPW = _B // _SC_NW
_NGRP = _GPW // 16
_BROWS = _N // 16
_SEARCH_STEPS = 19


def _sc_offsets_body(batch_hbm, out_hbm, idx_v, probe_v, res_v, sem):
    wid = lax.axis_index("s") * _SC_NC + lax.axis_index("c")
    base = wid * _GPW
    iota = lax.iota(jnp.int32, 16)
    gids = [base + 16 * k + iota for k in range(_NGRP)]

    def step(_, carry):
        lors = carry[:_NGRP]
        hirs = carry[_NGRP:]
        mids = [lax.shift_right_logical(lo + hi, 1)
                for lo, hi in zip(lors, hirs)]
        for k in range(_NGRP):
            idx_v[pl.ds(16 * k, 16)] = mids[k]
        pltpu.async_copy(batch_hbm.at[idx_v], probe_v, sem).wait()
        new_lo, new_hi = [], []
        for k in range(_NGRP):
            vals = probe_v[pl.ds(16 * k, 16)]
            pred = vals < gids[k]
            new_lo.append(lax.select(pred, mids[k] + 1, lors[k]))
            new_hi.append(lax.select(pred, hirs[k], mids[k]))
        return tuple(new_lo) + tuple(new_hi)

    zero = jnp.zeros((16,), jnp.int32)
    init = tuple(zero for _ in range(_NGRP)) + \
        tuple(jnp.full((16,), _N, jnp.int32) for _ in range(_NGRP))
    final = lax.fori_loop(0, _SEARCH_STEPS, step, init)
    for k in range(_NGRP):
        res_v[pl.ds(16 * k, 16)] = jnp.minimum(final[k], _N - _ROWS)
    pltpu.sync_copy(res_v, out_hbm.at[pl.ds(base, _GPW)])


def _sc_offsets(batch2):
    mesh = plsc.VectorSubcoreMesh(core_axis_name="c", subcore_axis_name="s")
    run = functools.partial(
        pl.kernel,
        mesh=mesh,
        out_type=jax.ShapeDtypeStruct((_B,), jnp.int32),
        scratch_types=[
            pltpu.VMEM((_GPW,), jnp.int32),
            pltpu.VMEM((_GPW,), jnp.int32),
            pltpu.VMEM((_GPW,), jnp.int32),
            pltpu.SemaphoreType.DMA,
        ],
    )(_sc_offsets_body)
    return run(batch2)


def _tc_body(offs_ref, emb_hbm, w1_ref, b1_ref, w2_ref, b2_ref,
             ln1w_ref, ln1b_ref, ln2w_ref, ln2b_ref,
             out_ref, ebuf, sem):
    blk = pl.program_id(0)
    slot = lax.rem(blk, 2)

    def fire(b, s):
        for g in range(_G):
            off = offs_ref[b * _G + g]
            pltpu.make_async_copy(
                emb_hbm.at[pl.ds(off, _ROWS)],
                ebuf.at[s, pl.ds(g * _ROWS, _ROWS)],
                sem.at[s],
            ).start()

    @pl.when(blk == 0)
    def _():
        fire(blk, slot)

    @pl.when(blk + 1 < _NBLK)
    def _():
        fire(blk + 1, lax.rem(blk + 1, 2))

    pltpu.make_async_copy(
        emb_hbm.at[pl.ds(0, _G * _ROWS)], ebuf.at[slot], sem.at[slot]
    ).wait()

    e = ebuf[slot]
    c = e.reshape(_G * _NCHUNK, _CHUNK, _D)
    csum = jnp.sum(c, axis=1).reshape(_G, _NCHUNK, _D)
    cmax = jnp.max(c, axis=1).reshape(_G, _NCHUNK, _D)

    for h in range(_NUM_HEADS):
        s = jnp.sum(csum[:, h:h + _WCHUNKS, :], axis=1)
        m = jnp.max(cmax[:, h:h + _WCHUNKS, :], axis=1)
        x = jnp.concatenate([s * (1.0 / 32.0), m, s], axis=-1)

        y = jnp.dot(x, w1_ref[h], preferred_element_type=jnp.float32)
        y = y + b1_ref[h:h + 1, :]
        mu = jnp.mean(y, axis=-1, keepdims=True)
        var = jnp.mean(jnp.square(y - mu), axis=-1, keepdims=True)
        y = (y - mu) * lax.rsqrt(var + _EPS) * ln1w_ref[...] + ln1b_ref[...]
        y = jnp.maximum(y, 0.0)

        z = jnp.dot(y, w2_ref[h], preferred_element_type=jnp.float32)
        z = z + b2_ref[h:h + 1, :]
        mu2 = jnp.mean(z, axis=-1, keepdims=True)
        var2 = jnp.mean(jnp.square(z - mu2), axis=-1, keepdims=True)
        z = (z - mu2) * lax.rsqrt(var2 + _EPS) * ln2w_ref[...] + ln2b_ref[...]
        out_ref[:, h, :] = jnp.maximum(z, 0.0)


def _run_tc(offsets, node_embeddings, proj1_weights, proj1_bias,
            proj2_weights, proj2_bias, ln1w, ln1b, ln2w, ln2b):
    full = lambda shape: pl.BlockSpec(shape, lambda i: (0,) * len(shape))
    return pl.pallas_call(
        _tc_body,
        grid=(_NBLK,),
        in_specs=[
            pl.BlockSpec(memory_space=pltpu.SMEM),
            pl.BlockSpec(memory_space=pltpu.HBM),
            full((_NUM_HEADS, 3 * _D, _H1)),
            full((_NUM_HEADS, _H1)),
            full((_NUM_HEADS, _H1, _H2)),
            full((_NUM_HEADS, _H2)),
            full((1, _H1)),
            full((1, _H1)),
            full((1, _H2)),
            full((1, _H2)),
        ],
        out_specs=pl.BlockSpec((_G, _NUM_HEADS, _H2), lambda i: (i, 0, 0)),
        out_shape=jax.ShapeDtypeStruct((_B, _NUM_HEADS, _H2), jnp.float32),
        scratch_shapes=[
            pltpu.VMEM((2, _G * _ROWS, _D), jnp.float32),
            pltpu.SemaphoreType.DMA((2,)),
        ],
        compiler_params=pltpu.CompilerParams(
            dimension_semantics=("arbitrary",),
        ),
    )(offsets, node_embeddings, proj1_weights, proj1_bias,
      proj2_weights, proj2_bias, ln1w, ln1b, ln2w, ln2b)


def kernel(node_embeddings, proj1_weights, proj1_bias, ln1_weight, ln1_bias,
           proj2_weights, proj2_bias, ln2_weight, ln2_bias, edge_index, batch):
    del edge_index
    offsets = _sc_offsets(batch)
    return _run_tc(
        offsets, node_embeddings, proj1_weights, proj1_bias,
        proj2_weights, proj2_bias,
        ln1_weight.reshape(1, _H1), ln1_bias.reshape(1, _H1),
        ln2_weight.reshape(1, _H2), ln2_bias.reshape(1, _H2),
    )


# speedup vs baseline: 22.1933x; 1.3879x over previous
import functools

import numpy as np
import jax
import jax.numpy as jnp
from jax import lax
from jax.experimental import pallas as pl
from jax.experimental.pallas import tpu as pltpu
from jax.experimental.pallas import tpu_sc as plsc

_NUM_HEADS = 8
_D = 128
_ROWS = 88
_CHUNK = 8
_NCHUNK = _ROWS // _CHUNK
_WCHUNKS = 4
_B = 2048
_N = 524288
_G = 64
_NBLK = _B // _G
_H1 = 256
_H2 = 128
_EPS = 1e-5


def _window_matrix():
    w = np.zeros((_G * _NUM_HEADS, _G * _ROWS), np.float32)
    for g in range(_G):
        for h in range(_NUM_HEADS):
            lo = g * _ROWS + _CHUNK * h
            w[g * _NUM_HEADS + h, lo:lo + 32] = 1.0
    return w


_WBIG = _window_matrix()


_SC_NC = 2
_SC_NS = 16
_SC_NW = _SC_NC * _SC_NS
_GPW = _B // _SC_NW
_NGRP = _GPW // 16
_BROWS = _N // 16
_SEARCH_STEPS = 19


def _sc_offsets_body(batch_hbm, out_hbm, idx_v, probe_v, res_v, sem):
    wid = lax.axis_index("s") * _SC_NC + lax.axis_index("c")
    base = wid * _GPW
    iota = lax.iota(jnp.int32, 16)
    gids = [base + 16 * k + iota for k in range(_NGRP)]

    def step(_, carry):
        lors = carry[:_NGRP]
        hirs = carry[_NGRP:]
        mids = [lax.shift_right_logical(lo + hi, 1)
                for lo, hi in zip(lors, hirs)]
        for k in range(_NGRP):
            idx_v[pl.ds(16 * k, 16)] = mids[k]
        pltpu.async_copy(batch_hbm.at[idx_v], probe_v, sem).wait()
        new_lo, new_hi = [], []
        for k in range(_NGRP):
            vals = probe_v[pl.ds(16 * k, 16)]
            pred = vals < gids[k]
            new_lo.append(lax.select(pred, mids[k] + 1, lors[k]))
            new_hi.append(lax.select(pred, hirs[k], mids[k]))
        return tuple(new_lo) + tuple(new_hi)

    zero = jnp.zeros((16,), jnp.int32)
    init = tuple(zero for _ in range(_NGRP)) + \
        tuple(jnp.full((16,), _N, jnp.int32) for _ in range(_NGRP))
    final = lax.fori_loop(0, _SEARCH_STEPS, step, init)
    for k in range(_NGRP):
        res_v[pl.ds(16 * k, 16)] = jnp.minimum(final[k], _N - _ROWS)
    pltpu.sync_copy(res_v, out_hbm.at[pl.ds(base, _GPW)])


def _sc_offsets(batch2):
    mesh = plsc.VectorSubcoreMesh(core_axis_name="c", subcore_axis_name="s")
    run = functools.partial(
        pl.kernel,
        mesh=mesh,
        out_type=jax.ShapeDtypeStruct((_B,), jnp.int32),
        scratch_types=[
            pltpu.VMEM((_GPW,), jnp.int32),
            pltpu.VMEM((_GPW,), jnp.int32),
            pltpu.VMEM((_GPW,), jnp.int32),
            pltpu.SemaphoreType.DMA,
        ],
    )(_sc_offsets_body)
    return run(batch2)


def _tc_body(offs_ref, emb_hbm, wbig_ref, w1s_ref, w1m_ref, b1_ref,
             w2_ref, b2_ref, ln1w_ref, ln1b_ref, ln2w_ref, ln2b_ref,
             out_ref, ebuf, sem):
    blk = pl.program_id(0)
    slot = lax.rem(blk, 2)

    def fire(b, s):
        for g in range(_G):
            off = offs_ref[b * _G + g]
            pltpu.make_async_copy(
                emb_hbm.at[pl.ds(off, _ROWS)],
                ebuf.at[s, pl.ds(g * _ROWS, _ROWS)],
                sem.at[s],
            ).start()

    @pl.when(blk == 0)
    def _():
        fire(blk, slot)

    pltpu.make_async_copy(
        emb_hbm.at[pl.ds(0, _G * _ROWS)], ebuf.at[slot], sem.at[slot]
    ).wait()

    @pl.when(blk + 1 < _NBLK)
    def _():
        fire(blk + 1, lax.rem(blk + 1, 2))

    e = ebuf[slot]
    s_all = jnp.dot(wbig_ref[...], e,
                    preferred_element_type=jnp.float32)
    s3 = s_all.reshape(_G, _NUM_HEADS, _D)
    c = e.reshape(_G * _NCHUNK, _CHUNK, _D)
    cmax = jnp.max(c, axis=1).reshape(_G, _NCHUNK, _D)
    pmax = jnp.maximum(cmax[:, 0:_NCHUNK - 1], cmax[:, 1:_NCHUNK])

    ys = []
    for h in range(_NUM_HEADS):
        s = s3[:, h]
        m = jnp.maximum(pmax[:, h], pmax[:, h + 2])
        y = jnp.dot(s, w1s_ref[h], preferred_element_type=jnp.float32)
        y = y + jnp.dot(m, w1m_ref[h], preferred_element_type=jnp.float32)
        ys.append(y + b1_ref[h:h + 1, :])
    ya = jnp.concatenate(ys, axis=0)
    mu = jnp.mean(ya, axis=-1, keepdims=True)
    var = jnp.mean(jnp.square(ya - mu), axis=-1, keepdims=True)
    ya = (ya - mu) * lax.rsqrt(var + _EPS) * ln1w_ref[...] + ln1b_ref[...]
    ya = jnp.maximum(ya, 0.0)

    zs = []
    for h in range(_NUM_HEADS):
        z = jnp.dot(ya[h * _G:(h + 1) * _G], w2_ref[h],
                    preferred_element_type=jnp.float32)
        zs.append(z + b2_ref[h:h + 1, :])
    za = jnp.concatenate(zs, axis=0)
    mu2 = jnp.mean(za, axis=-1, keepdims=True)
    var2 = jnp.mean(jnp.square(za - mu2), axis=-1, keepdims=True)
    za = (za - mu2) * lax.rsqrt(var2 + _EPS) * ln2w_ref[...] + ln2b_ref[...]
    za = jnp.maximum(za, 0.0)
    for h in range(_NUM_HEADS):
        out_ref[:, h, :] = za[h * _G:(h + 1) * _G]


def _run_tc(offsets, node_embeddings, wbig, w1s, w1m, proj1_bias,
            proj2_weights, proj2_bias, ln1w, ln1b, ln2w, ln2b):
    full = lambda shape: pl.BlockSpec(shape, lambda i: (0,) * len(shape))
    return pl.pallas_call(
        _tc_body,
        grid=(_NBLK,),
        in_specs=[
            pl.BlockSpec(memory_space=pltpu.SMEM),
            pl.BlockSpec(memory_space=pltpu.HBM),
            full((_G * _NUM_HEADS, _G * _ROWS)),
            full((_NUM_HEADS, _D, _H1)),
            full((_NUM_HEADS, _D, _H1)),
            full((_NUM_HEADS, _H1)),
            full((_NUM_HEADS, _H1, _H2)),
            full((_NUM_HEADS, _H2)),
            full((1, _H1)),
            full((1, _H1)),
            full((1, _H2)),
            full((1, _H2)),
        ],
        out_specs=pl.BlockSpec((_G, _NUM_HEADS, _H2), lambda i: (i, 0, 0)),
        out_shape=jax.ShapeDtypeStruct((_B, _NUM_HEADS, _H2), jnp.float32),
        scratch_shapes=[
            pltpu.VMEM((2, _G * _ROWS, _D), jnp.float32),
            pltpu.SemaphoreType.DMA((2,)),
        ],
        compiler_params=pltpu.CompilerParams(
            dimension_semantics=("arbitrary",),
        ),
    )(offsets, node_embeddings, wbig, w1s, w1m, proj1_bias,
      proj2_weights, proj2_bias, ln1w, ln1b, ln2w, ln2b)


def kernel(node_embeddings, proj1_weights, proj1_bias, ln1_weight, ln1_bias,
           proj2_weights, proj2_bias, ln2_weight, ln2_bias, edge_index, batch):
    del edge_index
    offsets = _sc_offsets(batch)
    w1s = proj1_weights[:, :_D, :] * (1.0 / 32.0) + proj1_weights[:, 2 * _D:, :]
    w1m = proj1_weights[:, _D:2 * _D, :]
    return _run_tc(
        offsets, node_embeddings, jnp.asarray(_WBIG), w1s, w1m, proj1_bias,
        proj2_weights, proj2_bias,
        ln1_weight.reshape(1, _H1), ln1_bias.reshape(1, _H1),
        ln2_weight.reshape(1, _H2), ln2_bias.reshape(1, _H2),
    )


# speedup vs baseline: 26.2947x; 1.1848x over previous
import functools

import jax
import jax.numpy as jnp
from jax import lax
from jax.experimental import pallas as pl
from jax.experimental.pallas import tpu as pltpu
from jax.experimental.pallas import tpu_sc as plsc

_NUM_HEADS = 8
_D = 128
_ROWS = 88
_CHUNK = 8
_NCHUNK = _ROWS // _CHUNK
_WCHUNKS = 4
_B = 2048
_N = 524288
_G = 64
_NBLK = _B // _G
_H1 = 256
_H2 = 128
_EPS = 1e-5


_SC_NC = 2
_SC_NS = 16
_SC_NW = _SC_NC * _SC_NS
_GPW = _B // _SC_NW
_NGRP = _GPW // 16
_NPROBE = 16
_SSTEPS = [32768, 2048, 128, 8, 1]


def _sc_offsets_body(batch_hbm, out_hbm, idx_v, probe_v, res_v, sem):
    wid = lax.axis_index("s") * _SC_NC + lax.axis_index("c")
    base = wid * _GPW
    iota = lax.iota(jnp.int32, 16)
    gids = [base + 16 * k + iota for k in range(_NGRP)]

    los = [jnp.zeros((16,), jnp.int32) for _ in range(_NGRP)]
    for s in _SSTEPS:
        for i in range(_NPROBE):
            for k in range(_NGRP):
                row, col = divmod(i * _GPW + 16 * k, 128)
                idx_v[row, pl.ds(col, 16)] = jnp.minimum(los[k] + i * s,
                                                         _N - 1)
        copies = [pltpu.async_copy(batch_hbm.at[idx_v.at[m]], probe_v.at[m],
                                   sem) for m in range(8)]
        for cp in copies:
            cp.wait()
        new_los = []
        for k in range(_NGRP):
            cnt = jnp.zeros((16,), jnp.int32)
            for i in range(_NPROBE):
                row, col = divmod(i * _GPW + 16 * k, 128)
                vals = probe_v[row, pl.ds(col, 16)]
                cnt = cnt + jnp.where(vals < gids[k], 1, 0)
            adv = jnp.maximum(cnt - 1, 0) * s + jnp.minimum(cnt, 1)
            new_los.append(jnp.minimum(los[k] + adv, _N))
        los = new_los
    for k in range(_NGRP):
        res_v[pl.ds(16 * k, 16)] = jnp.minimum(los[k], _N - _ROWS)
    pltpu.sync_copy(res_v, out_hbm.at[pl.ds(base, _GPW)])


def _sc_offsets(batch2):
    mesh = plsc.VectorSubcoreMesh(core_axis_name="c", subcore_axis_name="s")
    run = functools.partial(
        pl.kernel,
        mesh=mesh,
        out_type=jax.ShapeDtypeStruct((_B,), jnp.int32),
        scratch_types=[
            pltpu.VMEM((8, 128), jnp.int32),
            pltpu.VMEM((8, 128), jnp.int32),
            pltpu.VMEM((_GPW,), jnp.int32),
            pltpu.SemaphoreType.DMA,
        ],
    )(_sc_offsets_body)
    return run(batch2)


def _tc_body(offs_ref, emb_hbm, w1s_ref, w1m_ref, b1_ref,
             w2_ref, b2_ref, ln1w_ref, ln1b_ref, ln2w_ref, ln2b_ref,
             out_ref, ebuf, sem):
    blk = pl.program_id(0)
    slot = lax.rem(blk, 2)

    def fire(b, s):
        for g in range(_G):
            off = offs_ref[b * _G + g]
            pltpu.make_async_copy(
                emb_hbm.at[pl.ds(off, _ROWS)],
                ebuf.at[s, pl.ds(g * _ROWS, _ROWS)],
                sem.at[s],
            ).start()

    @pl.when(blk == 0)
    def _():
        fire(blk, slot)

    pltpu.make_async_copy(
        emb_hbm.at[pl.ds(0, _G * _ROWS)], ebuf.at[slot], sem.at[slot]
    ).wait()

    @pl.when(blk + 1 < _NBLK)
    def _():
        fire(blk + 1, lax.rem(blk + 1, 2))

    e = ebuf[slot]
    c = e.reshape(_G * _NCHUNK, _CHUNK, _D)
    csum = jnp.sum(c, axis=1).reshape(_G, _NCHUNK, _D)
    cmax = jnp.max(c, axis=1).reshape(_G, _NCHUNK, _D)
    psum = csum[:, 0:_NCHUNK - 1] + csum[:, 1:_NCHUNK]
    pmax = jnp.maximum(cmax[:, 0:_NCHUNK - 1], cmax[:, 1:_NCHUNK])

    ys = []
    for h in range(_NUM_HEADS):
        s = psum[:, h] + psum[:, h + 2]
        m = jnp.maximum(pmax[:, h], pmax[:, h + 2])
        y = jnp.dot(s, w1s_ref[h], preferred_element_type=jnp.float32)
        y = y + jnp.dot(m, w1m_ref[h], preferred_element_type=jnp.float32)
        ys.append(y + b1_ref[h:h + 1, :])
    ya = jnp.concatenate(ys, axis=0)
    mu = jnp.mean(ya, axis=-1, keepdims=True)
    var = jnp.mean(jnp.square(ya - mu), axis=-1, keepdims=True)
    ya = (ya - mu) * lax.rsqrt(var + _EPS) * ln1w_ref[...] + ln1b_ref[...]
    ya = jnp.maximum(ya, 0.0)

    zs = []
    for h in range(_NUM_HEADS):
        z = jnp.dot(ya[h * _G:(h + 1) * _G], w2_ref[h],
                    preferred_element_type=jnp.float32)
        zs.append(z + b2_ref[h:h + 1, :])
    za = jnp.concatenate(zs, axis=0)
    mu2 = jnp.mean(za, axis=-1, keepdims=True)
    var2 = jnp.mean(jnp.square(za - mu2), axis=-1, keepdims=True)
    za = (za - mu2) * lax.rsqrt(var2 + _EPS) * ln2w_ref[...] + ln2b_ref[...]
    za = jnp.maximum(za, 0.0)
    for h in range(_NUM_HEADS):
        out_ref[:, h, :] = za[h * _G:(h + 1) * _G]


def _run_tc(offsets, node_embeddings, w1s, w1m, proj1_bias,
            proj2_weights, proj2_bias, ln1w, ln1b, ln2w, ln2b):
    full = lambda shape: pl.BlockSpec(shape, lambda i: (0,) * len(shape))
    return pl.pallas_call(
        _tc_body,
        grid=(_NBLK,),
        in_specs=[
            pl.BlockSpec(memory_space=pltpu.SMEM),
            pl.BlockSpec(memory_space=pltpu.HBM),
            full((_NUM_HEADS, _D, _H1)),
            full((_NUM_HEADS, _D, _H1)),
            full((_NUM_HEADS, _H1)),
            full((_NUM_HEADS, _H1, _H2)),
            full((_NUM_HEADS, _H2)),
            full((1, _H1)),
            full((1, _H1)),
            full((1, _H2)),
            full((1, _H2)),
        ],
        out_specs=pl.BlockSpec((_G, _NUM_HEADS, _H2), lambda i: (i, 0, 0)),
        out_shape=jax.ShapeDtypeStruct((_B, _NUM_HEADS, _H2), jnp.float32),
        scratch_shapes=[
            pltpu.VMEM((2, _G * _ROWS, _D), jnp.float32),
            pltpu.SemaphoreType.DMA((2,)),
        ],
        compiler_params=pltpu.CompilerParams(
            dimension_semantics=("arbitrary",),
        ),
    )(offsets, node_embeddings, w1s, w1m, proj1_bias,
      proj2_weights, proj2_bias, ln1w, ln1b, ln2w, ln2b)


def kernel(node_embeddings, proj1_weights, proj1_bias, ln1_weight, ln1_bias,
           proj2_weights, proj2_bias, ln2_weight, ln2_bias, edge_index, batch):
    del edge_index
    offsets = _sc_offsets(batch)
    w1s = proj1_weights[:, :_D, :] * (1.0 / 32.0) + proj1_weights[:, 2 * _D:, :]
    w1m = proj1_weights[:, _D:2 * _D, :]
    return _run_tc(
        offsets, node_embeddings, w1s, w1m, proj1_bias,
        proj2_weights, proj2_bias,
        ln1_weight.reshape(1, _H1), ln1_bias.reshape(1, _H1),
        ln2_weight.reshape(1, _H2), ln2_bias.reshape(1, _H2),
    )
